# Initial kernel scaffold; baseline (speedup 1.0000x reference)
import jax
import jax.numpy as jnp
from jax.experimental import pallas as pl


def kernel(x, logits, a_idx, b_idx):
    raise NotImplementedError("write your pallas kernel here")


# speedup vs baseline: 4.2232x; 4.2232x over previous
import functools

import jax
import jax.numpy as jnp
import numpy as np
from jax import lax
from jax.experimental import pallas as pl
from jax.experimental.pallas import tpu as pltpu
from jax.experimental.pallas import tpu_sc as plsc

_IN_DIM = 8192
_OUT_DIM = 16384
_BATCH = 1024
_L = 16

_GATE_COEF = np.zeros((16, 16), np.float32)
_GATE_COEF[:, :4] = [
    [0, 0, 0, 0],
    [0, 0, 0, 1],
    [0, 1, 0, -1],
    [0, 1, 0, 0],
    [0, 0, 1, -1],
    [0, 0, 1, 0],
    [0, 1, 1, -2],
    [0, 1, 1, -1],
    [1, -1, -1, 1],
    [1, -1, -1, 2],
    [1, 0, -1, 0],
    [1, 0, -1, 1],
    [1, -1, 0, 0],
    [1, -1, 0, 1],
    [1, 0, 0, -1],
    [1, 0, 0, 0],
]


def _coef_body(l_ref, c_ref, w_ref):
    l = l_ref[...]
    m = jnp.max(l, axis=-1, keepdims=True)
    e = jnp.exp(l - m)
    p = e / jnp.sum(e, axis=-1, keepdims=True)
    w_ref[...] = lax.dot_general(
        p, c_ref[...], (((1,), (0,)), ((), ())),
        preferred_element_type=jnp.float32)


def _coefs(logits):
    return pl.pallas_call(
        _coef_body,
        out_shape=jax.ShapeDtypeStruct((_OUT_DIM, 16), jnp.float32),
    )(logits, jnp.asarray(_GATE_COEF))


_NW = 32
_GPW = _OUT_DIM // _NW
_G = 16
_NCHUNK = _GPW // _G
_LCHUNKS = _BATCH // _L


def _sc_body(xt_hbm, w_hbm, aidx_hbm, bidx_hbm, out_hbm,
             aidx_v, bidx_v, w_v, rows_a, rows_b, out_v, sem_a, sem_b):
    wid = lax.axis_index("s") * 2 + lax.axis_index("c")

    def chunk_body(i, carry):
        base = wid * _GPW + i * _G
        pltpu.sync_copy(aidx_hbm.at[pl.ds(base, _G)], aidx_v)
        pltpu.sync_copy(bidx_hbm.at[pl.ds(base, _G)], bidx_v)
        pltpu.sync_copy(w_hbm.at[pl.ds(base, _G)], w_v)
        ca = pltpu.async_copy(xt_hbm.at[aidx_v], rows_a, sem_a)
        cb = pltpu.async_copy(xt_hbm.at[bidx_v], rows_b, sem_b)
        ca.wait()
        cb.wait()
        for g in range(_G):
            wrow = w_v[g, pl.ds(0, _L)]
            w1 = wrow[0]
            wa = wrow[1]
            wb = wrow[2]
            wab = wrow[3]

            def lane_body(c, inner, g=g, w1=w1, wa=wa, wb=wb, wab=wab):
                a = rows_a[g, pl.ds(c * _L, _L)]
                b = rows_b[g, pl.ds(c * _L, _L)]
                out_v[g, pl.ds(c * _L, _L)] = w1 + wa * a + wb * b + wab * (a * b)
                return inner

            lax.fori_loop(0, _LCHUNKS, lane_body, 0)
        pltpu.sync_copy(out_v, out_hbm.at[pl.ds(base, _G)])
        return carry

    lax.fori_loop(0, _NCHUNK, chunk_body, 0)


_sc_call = functools.partial(
    pl.kernel,
    mesh=plsc.VectorSubcoreMesh(core_axis_name="c", subcore_axis_name="s"),
    out_type=jax.ShapeDtypeStruct((_OUT_DIM, _BATCH), jnp.float32),
    scratch_types=[
        pltpu.VMEM((_G,), jnp.int32),
        pltpu.VMEM((_G,), jnp.int32),
        pltpu.VMEM((_G, 16), jnp.float32),
        pltpu.VMEM((_G, _BATCH), jnp.float32),
        pltpu.VMEM((_G, _BATCH), jnp.float32),
        pltpu.VMEM((_G, _BATCH), jnp.float32),
        pltpu.SemaphoreType.DMA,
        pltpu.SemaphoreType.DMA,
    ],
)(_sc_body)


@jax.jit
def kernel(x, logits, a_idx, b_idx):
    xt = x.T
    w = _coefs(logits)
    yt = _sc_call(xt, w, a_idx, b_idx)
    return yt.T
